# baseline (speedup 1.0000x reference)
import functools

import jax
import jax.numpy as jnp
from jax import lax
from jax.experimental import pallas as pl
from jax.experimental.pallas import tpu as pltpu
from jax.experimental.pallas import tpu_sc as plsc

_NC = 2
_NS = 16
_B = 16
_Q = 300
_C = 92
_L = 16
_W = 160
_G = _W // _L


def _sc_body(logits_hbm, boxes_hbm, params_hbm, *rest):
    outs = rest[:_B]
    lslab, bslab, oslab, pvm = rest[_B:]
    wid = lax.axis_index("s") * _NC + lax.axis_index("c")
    b = wid // 2
    half = wid % 2

    pltpu.sync_copy(params_hbm, pvm)

    @pl.when(half == 0)
    def _():
        pltpu.sync_copy(logits_hbm.at[b, pl.ds(0, _W), :], lslab)
        pltpu.sync_copy(boxes_hbm.at[b, pl.ds(0, _W), :], bslab)

    @pl.when(half == 1)
    def _():
        pltpu.sync_copy(logits_hbm.at[b, pl.ds(144, 156), :], lslab.at[0:156, :])
        pltpu.sync_copy(boxes_hbm.at[b, pl.ds(144, 156), :], bslab.at[0:156, :])

    img_h = pvm[b, 0, :]
    img_w = pvm[b, 1, :]
    conf = pvm[b, 2, :]
    zeros = jnp.zeros((_L,), jnp.float32)
    lane = lax.iota(jnp.int32, _L)

    def group(g, carry):
        q = g * _L + lane

        def p1(c, acc):
            m, lbl = acc
            v = plsc.load_gather(lslab, [q, jnp.full((_L,), c, jnp.int32)])
            upd = v > m
            lbl = jnp.where(upd, jnp.full((_L,), c).astype(jnp.float32), lbl)
            return jnp.maximum(m, v), lbl

        m, lbl = lax.fori_loop(
            0, _C - 1, p1, (jnp.full((_L,), -jnp.inf, jnp.float32), zeros)
        )
        v91 = plsc.load_gather(lslab, [q, jnp.full((_L,), _C - 1, jnp.int32)])
        mall = jnp.maximum(m, v91)

        def p2(c, s):
            v = plsc.load_gather(lslab, [q, jnp.full((_L,), c, jnp.int32)])
            return s + jnp.exp(v - mall)

        s = lax.fori_loop(0, _C, p2, zeros)
        score = jnp.exp(m - mall) / s

        cx = plsc.load_gather(bslab, [q, jnp.full((_L,), 0, jnp.int32)])
        cy = plsc.load_gather(bslab, [q, jnp.full((_L,), 1, jnp.int32)])
        w = plsc.load_gather(bslab, [q, jnp.full((_L,), 2, jnp.int32)])
        h = plsc.load_gather(bslab, [q, jnp.full((_L,), 3, jnp.int32)])
        y1 = (cy - 0.5 * h) * img_h
        x1 = (cx - 0.5 * w) * img_w
        y2 = (cy + 0.5 * h) * img_h
        x2 = (cx + 0.5 * w) * img_w

        keep = score > conf
        for j, val in enumerate((y1, x1, y2, x2, score, lbl)):
            plsc.store_scatter(
                oslab,
                [q, jnp.full((_L,), j, jnp.int32)],
                jnp.where(keep, val, zeros),
            )
        return carry

    lax.fori_loop(0, _G, group, 0)

    for i in range(_B):
        @pl.when(jnp.logical_and(b == i, half == 0))
        def _(out_i=outs[i]):
            pltpu.sync_copy(oslab, out_i.at[pl.ds(0, _W), :])

        @pl.when(jnp.logical_and(b == i, half == 1))
        def _(out_i=outs[i]):
            pltpu.sync_copy(
                oslab.at[pl.ds(16, 140), :], out_i.at[pl.ds(160, 140), :]
            )


_sc_decode = functools.partial(
    pl.kernel,
    mesh=plsc.VectorSubcoreMesh(core_axis_name="c", subcore_axis_name="s"),
    out_type=tuple(
        jax.ShapeDtypeStruct((_Q, 6), jnp.float32) for _ in range(_B)
    ),
    compiler_params=pltpu.CompilerParams(
        use_tc_tiling_on_sc=False, needs_layout_passes=False
    ),
    scratch_types=[
        pltpu.VMEM((_W, _C), jnp.float32),
        pltpu.VMEM((_W, 4), jnp.float32),
        pltpu.VMEM((_W, 6), jnp.float32),
        pltpu.VMEM((_B, 3, _L), jnp.float32),
    ],
)(_sc_body)


def kernel(pred_logits, pred_boxes, target_sizes, confidence):
    conf_col = jnp.broadcast_to(
        jnp.asarray(confidence, jnp.float32).reshape(1, 1), (_B, 1)
    )
    params = jnp.concatenate([target_sizes.astype(jnp.float32), conf_col], axis=1)
    params = jnp.broadcast_to(params[:, :, None], (_B, 3, _L))
    return _sc_decode(pred_logits, pred_boxes, params)


# speedup vs baseline: 1.0272x; 1.0272x over previous
import functools

import jax
import jax.numpy as jnp
from jax import lax
from jax.experimental import pallas as pl
from jax.experimental.pallas import tpu as pltpu
from jax.experimental.pallas import tpu_sc as plsc

_NC = 2
_NS = 16
_B = 16
_Q = 300
_QP = 304
_C = 92
_L = 16
_W = 160
_G = _W // _L
_U = 4


def _combine(ma, la, mb, lb):
    m = jnp.maximum(ma, mb)
    l = jnp.where(mb > ma, lb, la)
    return m, jnp.where(mb == ma, jnp.minimum(la, lb), l)


def _sc_body(logits_hbm, sclb_hbm, lslab, sslab, llab):
    wid = lax.axis_index("s") * _NC + lax.axis_index("c")
    b = wid // 2
    half = wid % 2

    @pl.when(half == 0)
    def _():
        pltpu.sync_copy(logits_hbm.at[b, pl.ds(0, _W), :], lslab)

    @pl.when(half == 1)
    def _():
        pltpu.sync_copy(logits_hbm.at[b, pl.ds(144, 156), :], lslab.at[0:156, :])

    zeros = jnp.zeros((_L,), jnp.float32)
    neg = jnp.full((_L,), -jnp.inf, jnp.float32)
    lane = lax.iota(jnp.int32, _L)

    def group(g, carry):
        q = g * _L + lane

        def p1(i, acc):
            m0, m1, m2, m3, l0, l1, l2, l3, ci, cf = acc
            v0 = plsc.load_gather(lslab, [q, ci])
            v1 = plsc.load_gather(lslab, [q, ci + 1])
            v2 = plsc.load_gather(lslab, [q, ci + 2])
            v3 = plsc.load_gather(lslab, [q, ci + 3])
            l0 = jnp.where(v0 > m0, cf, l0)
            l1 = jnp.where(v1 > m1, cf + 1.0, l1)
            l2 = jnp.where(v2 > m2, cf + 2.0, l2)
            l3 = jnp.where(v3 > m3, cf + 3.0, l3)
            return (
                jnp.maximum(m0, v0), jnp.maximum(m1, v1),
                jnp.maximum(m2, v2), jnp.maximum(m3, v3),
                l0, l1, l2, l3, ci + _U, cf + 4.0,
            )

        init = (neg, neg, neg, neg, zeros, zeros, zeros, zeros,
                jnp.zeros((_L,), jnp.int32), zeros)
        m0, m1, m2, m3, l0, l1, l2, l3, ci, cf = lax.fori_loop(0, 22, p1, init)
        v0 = plsc.load_gather(lslab, [q, ci])
        v1 = plsc.load_gather(lslab, [q, ci + 1])
        v2 = plsc.load_gather(lslab, [q, ci + 2])
        l0 = jnp.where(v0 > m0, cf, l0)
        l1 = jnp.where(v1 > m1, cf + 1.0, l1)
        l2 = jnp.where(v2 > m2, cf + 2.0, l2)
        m0 = jnp.maximum(m0, v0)
        m1 = jnp.maximum(m1, v1)
        m2 = jnp.maximum(m2, v2)
        ma, la = _combine(m0, l0, m1, l1)
        mb, lb = _combine(m2, l2, m3, l3)
        m91, lbl = _combine(ma, la, mb, lb)
        v91 = plsc.load_gather(lslab, [q, jnp.full((_L,), _C - 1, jnp.int32)])
        mall = jnp.maximum(m91, v91)

        def p2(i, acc):
            s0, s1, s2, s3, ci2 = acc
            v0 = plsc.load_gather(lslab, [q, ci2])
            v1 = plsc.load_gather(lslab, [q, ci2 + 1])
            v2 = plsc.load_gather(lslab, [q, ci2 + 2])
            v3 = plsc.load_gather(lslab, [q, ci2 + 3])
            return (
                s0 + jnp.exp(v0 - mall), s1 + jnp.exp(v1 - mall),
                s2 + jnp.exp(v2 - mall), s3 + jnp.exp(v3 - mall),
                ci2 + _U,
            )

        init2 = (zeros, zeros, zeros, zeros, jnp.zeros((_L,), jnp.int32))
        s0, s1, s2, s3, ci2 = lax.fori_loop(0, 23, p2, init2)
        s = (s0 + s1) + (s2 + s3)
        score = jnp.exp(m91 - mall) / s

        sslab[pl.ds(g * _L, _L)] = score
        llab[pl.ds(g * _L, _L)] = lbl
        return carry

    lax.fori_loop(0, _G, group, 0)

    @pl.when(half == 0)
    def _():
        pltpu.sync_copy(sslab, sclb_hbm.at[0, b, pl.ds(0, _W)])
        pltpu.sync_copy(llab, sclb_hbm.at[1, b, pl.ds(0, _W)])

    @pl.when(half == 1)
    def _():
        pltpu.sync_copy(sslab, sclb_hbm.at[0, b, pl.ds(144, _W)])
        pltpu.sync_copy(llab, sclb_hbm.at[1, b, pl.ds(144, _W)])


_sc_softmax = functools.partial(
    pl.kernel,
    mesh=plsc.VectorSubcoreMesh(core_axis_name="c", subcore_axis_name="s"),
    out_type=jax.ShapeDtypeStruct((2, _B, _QP), jnp.float32),
    compiler_params=pltpu.CompilerParams(
        use_tc_tiling_on_sc=False, needs_layout_passes=False
    ),
    scratch_types=[
        pltpu.VMEM((_W, _C), jnp.float32),
        pltpu.VMEM((_W,), jnp.float32),
        pltpu.VMEM((_W,), jnp.float32),
    ],
)(_sc_body)


def _tc_body(boxes_ref, sclb_ref, ts_ref, conf_ref, *out_refs):
    boxes = boxes_ref[...]
    cx = boxes[:, :, 0:1]
    cy = boxes[:, :, 1:2]
    w = boxes[:, :, 2:3]
    h = boxes[:, :, 3:4]
    ts = ts_ref[...].astype(jnp.float32)
    img_h = ts[:, 0:1][:, :, None]
    img_w = ts[:, 1:2][:, :, None]
    y1 = (cy - 0.5 * h) * img_h
    x1 = (cx - 0.5 * w) * img_w
    y2 = (cy + 0.5 * h) * img_h
    x2 = (cx + 0.5 * w) * img_w
    sclb = sclb_ref[...]
    score = sclb[0][:, :_Q, None]
    label = sclb[1][:, :_Q, None]
    out = jnp.concatenate([y1, x1, y2, x2, score, label], axis=2)
    keep = score > conf_ref[0, 0]
    out = jnp.where(keep, out, 0.0)
    for i in range(_B):
        out_refs[i][...] = out[i]


def kernel(pred_logits, pred_boxes, target_sizes, confidence):
    sclb = _sc_softmax(pred_logits)
    conf = jnp.asarray(confidence, jnp.float32).reshape(1, 1)
    return pl.pallas_call(
        _tc_body,
        out_shape=tuple(
            jax.ShapeDtypeStruct((_Q, 6), jnp.float32) for _ in range(_B)
        ),
    )(pred_boxes, sclb, target_sizes, conf)


# speedup vs baseline: 1.4999x; 1.4603x over previous
import functools

import jax
import jax.numpy as jnp
from jax import lax
from jax.experimental import pallas as pl
from jax.experimental.pallas import tpu as pltpu
from jax.experimental.pallas import tpu_sc as plsc

_NC = 2
_NS = 16
_B = 16
_Q = 300
_QP = 304
_C = 92
_L = 16
_G = 10


def _combine(ma, la, mb, lb):
    m = jnp.maximum(ma, mb)
    l = jnp.where(mb > ma, lb, la)
    return m, jnp.where(mb == ma, jnp.minimum(la, lb), l)


def _sc_body(lt_hbm, sclb_hbm, lslab, sslab, llab):
    wid = lax.axis_index("s") * _NC + lax.axis_index("c")
    b = wid // 2
    half = wid % 2
    g0 = half * 9

    pltpu.sync_copy(lt_hbm.at[:, b, pl.ds(g0, _G), :], lslab)

    zeros = jnp.zeros((_L,), jnp.float32)
    neg = jnp.full((_L,), -jnp.inf, jnp.float32)

    def group(g, carry):
        def p1(i, acc):
            m0, m1, m2, m3, l0, l1, l2, l3, cf = acc
            c = i * 4
            v0 = lslab[c, g, :]
            v1 = lslab[c + 1, g, :]
            v2 = lslab[c + 2, g, :]
            v3 = lslab[c + 3, g, :]
            l0 = jnp.where(v0 > m0, cf, l0)
            l1 = jnp.where(v1 > m1, cf + 1.0, l1)
            l2 = jnp.where(v2 > m2, cf + 2.0, l2)
            l3 = jnp.where(v3 > m3, cf + 3.0, l3)
            return (
                jnp.maximum(m0, v0), jnp.maximum(m1, v1),
                jnp.maximum(m2, v2), jnp.maximum(m3, v3),
                l0, l1, l2, l3, cf + 4.0,
            )

        init = (neg, neg, neg, neg, zeros, zeros, zeros, zeros, zeros)
        m0, m1, m2, m3, l0, l1, l2, l3, cf = lax.fori_loop(0, 22, p1, init)
        v0 = lslab[88, g, :]
        v1 = lslab[89, g, :]
        v2 = lslab[90, g, :]
        l0 = jnp.where(v0 > m0, cf, l0)
        l1 = jnp.where(v1 > m1, cf + 1.0, l1)
        l2 = jnp.where(v2 > m2, cf + 2.0, l2)
        m0 = jnp.maximum(m0, v0)
        m1 = jnp.maximum(m1, v1)
        m2 = jnp.maximum(m2, v2)
        ma, la = _combine(m0, l0, m1, l1)
        mb, lb = _combine(m2, l2, m3, l3)
        m91, lbl = _combine(ma, la, mb, lb)
        mall = jnp.maximum(m91, lslab[91, g, :])

        def p2(i, acc):
            s0, s1, s2, s3 = acc
            c = i * 4
            return (
                s0 + jnp.exp(lslab[c, g, :] - mall),
                s1 + jnp.exp(lslab[c + 1, g, :] - mall),
                s2 + jnp.exp(lslab[c + 2, g, :] - mall),
                s3 + jnp.exp(lslab[c + 3, g, :] - mall),
            )

        s0, s1, s2, s3 = lax.fori_loop(0, 23, p2, (zeros, zeros, zeros, zeros))
        s = (s0 + s1) + (s2 + s3)
        sslab[g, :] = jnp.exp(m91 - mall) / s
        llab[g, :] = lbl
        return carry

    lax.fori_loop(0, _G, group, 0)

    pltpu.sync_copy(sslab, sclb_hbm.at[0, b, pl.ds(g0, _G), :])
    pltpu.sync_copy(llab, sclb_hbm.at[1, b, pl.ds(g0, _G), :])


_sc_softmax = functools.partial(
    pl.kernel,
    mesh=plsc.VectorSubcoreMesh(core_axis_name="c", subcore_axis_name="s"),
    out_type=jax.ShapeDtypeStruct((2, _B, _QP // _L, _L), jnp.float32),
    compiler_params=pltpu.CompilerParams(
        use_tc_tiling_on_sc=False, needs_layout_passes=False
    ),
    scratch_types=[
        pltpu.VMEM((_C, _G, _L), jnp.float32),
        pltpu.VMEM((_G, _L), jnp.float32),
        pltpu.VMEM((_G, _L), jnp.float32),
    ],
)(_sc_body)


def _tc_body(bt_ref, sclb_ref, tsf_ref, conf_ref, *out_refs):
    bt = bt_ref[...]
    tsf = tsf_ref[...]
    cx = bt[:, 0, :]
    cy = bt[:, 1, :]
    w = bt[:, 2, :]
    h = bt[:, 3, :]
    img_h = tsf[:, 0:1]
    img_w = tsf[:, 1:2]
    y1 = (cy - 0.5 * h) * img_h
    x1 = (cx - 0.5 * w) * img_w
    y2 = (cy + 0.5 * h) * img_h
    x2 = (cx + 0.5 * w) * img_w
    sclb = sclb_ref[...]
    sc = sclb[0][:, :_Q]
    lb = sclb[1][:, :_Q]
    keep = sc > conf_ref[0, 0]
    for i in range(_B):
        leaf = jnp.stack([y1[i], x1[i], y2[i], x2[i], sc[i], lb[i]], axis=0)
        out_refs[i][...] = jnp.where(keep[i][None, :], leaf, 0.0)


def kernel(pred_logits, pred_boxes, target_sizes, confidence):
    lt = jnp.transpose(pred_logits, (2, 0, 1))
    lt = jnp.pad(lt, ((0, 0), (0, 0), (0, _QP - _Q)))
    lt = lt.reshape(_C, _B, _QP // _L, _L)
    sclb = _sc_softmax(lt).reshape(2, _B, _QP)
    bt = jnp.transpose(pred_boxes, (0, 2, 1))
    tsf = target_sizes.astype(jnp.float32)
    conf = jnp.asarray(confidence, jnp.float32).reshape(1, 1)
    outs = pl.pallas_call(
        _tc_body,
        out_shape=tuple(
            jax.ShapeDtypeStruct((6, _Q), jnp.float32) for _ in range(_B)
        ),
    )(bt, sclb, tsf, conf)
    return tuple(jnp.transpose(o) for o in outs)


# speedup vs baseline: 2.5052x; 1.6702x over previous
import functools

import jax
import jax.numpy as jnp
from jax import lax
from jax.experimental import pallas as pl
from jax.experimental.pallas import tpu as pltpu
from jax.experimental.pallas import tpu_sc as plsc

_NC = 2
_NS = 16
_B = 16
_Q = 300
_QP = 304
_C = 92
_L = 16
_G = 10


def _combine(ma, la, mb, lb):
    m = jnp.maximum(ma, mb)
    l = jnp.where(mb > ma, lb, la)
    return m, jnp.where(mb == ma, jnp.minimum(la, lb), l)


def _sc_body(lt_hbm, sclb_hbm, lslab, sslab, llab):
    wid = lax.axis_index("s") * _NC + lax.axis_index("c")
    b = wid // 2
    half = wid % 2
    @pl.when(half == 0)
    def _():
        pltpu.sync_copy(lt_hbm.at[:, b, pl.ds(0, _G * _L)], lslab)

    @pl.when(half == 1)
    def _():
        pltpu.sync_copy(lt_hbm.at[:, b, pl.ds(144, _G * _L)], lslab)

    zeros = jnp.zeros((_L,), jnp.float32)
    neg = jnp.full((_L,), -jnp.inf, jnp.float32)

    def group(g, carry):
        qo = pl.multiple_of(g * _L, _L)

        def p1(i, acc):
            m0, m1, m2, m3, l0, l1, l2, l3, cf = acc
            c = i * 4
            v0 = lslab[c, pl.ds(qo, _L)]
            v1 = lslab[c + 1, pl.ds(qo, _L)]
            v2 = lslab[c + 2, pl.ds(qo, _L)]
            v3 = lslab[c + 3, pl.ds(qo, _L)]
            l0 = jnp.where(v0 > m0, cf, l0)
            l1 = jnp.where(v1 > m1, cf + 1.0, l1)
            l2 = jnp.where(v2 > m2, cf + 2.0, l2)
            l3 = jnp.where(v3 > m3, cf + 3.0, l3)
            return (
                jnp.maximum(m0, v0), jnp.maximum(m1, v1),
                jnp.maximum(m2, v2), jnp.maximum(m3, v3),
                l0, l1, l2, l3, cf + 4.0,
            )

        init = (neg, neg, neg, neg, zeros, zeros, zeros, zeros, zeros)
        m0, m1, m2, m3, l0, l1, l2, l3, cf = lax.fori_loop(0, 22, p1, init)
        v0 = lslab[88, pl.ds(qo, _L)]
        v1 = lslab[89, pl.ds(qo, _L)]
        v2 = lslab[90, pl.ds(qo, _L)]
        l0 = jnp.where(v0 > m0, cf, l0)
        l1 = jnp.where(v1 > m1, cf + 1.0, l1)
        l2 = jnp.where(v2 > m2, cf + 2.0, l2)
        m0 = jnp.maximum(m0, v0)
        m1 = jnp.maximum(m1, v1)
        m2 = jnp.maximum(m2, v2)
        ma, la = _combine(m0, l0, m1, l1)
        mb, lb = _combine(m2, l2, m3, l3)
        m91, lbl = _combine(ma, la, mb, lb)
        mall = jnp.maximum(m91, lslab[91, pl.ds(qo, _L)])

        def p2(i, acc):
            s0, s1, s2, s3 = acc
            c = i * 4
            return (
                s0 + jnp.exp(lslab[c, pl.ds(qo, _L)] - mall),
                s1 + jnp.exp(lslab[c + 1, pl.ds(qo, _L)] - mall),
                s2 + jnp.exp(lslab[c + 2, pl.ds(qo, _L)] - mall),
                s3 + jnp.exp(lslab[c + 3, pl.ds(qo, _L)] - mall),
            )

        s0, s1, s2, s3 = lax.fori_loop(0, 23, p2, (zeros, zeros, zeros, zeros))
        s = (s0 + s1) + (s2 + s3)
        sslab[pl.ds(qo, _L)] = jnp.exp(m91 - mall) / s
        llab[pl.ds(qo, _L)] = lbl
        return carry

    lax.fori_loop(0, _G, group, 0)

    @pl.when(half == 0)
    def _():
        pltpu.sync_copy(sslab, sclb_hbm.at[0, b, pl.ds(0, _G * _L)])
        pltpu.sync_copy(llab, sclb_hbm.at[1, b, pl.ds(0, _G * _L)])

    @pl.when(half == 1)
    def _():
        pltpu.sync_copy(sslab, sclb_hbm.at[0, b, pl.ds(144, _G * _L)])
        pltpu.sync_copy(llab, sclb_hbm.at[1, b, pl.ds(144, _G * _L)])


_sc_softmax = functools.partial(
    pl.kernel,
    mesh=plsc.VectorSubcoreMesh(core_axis_name="c", subcore_axis_name="s"),
    out_type=jax.ShapeDtypeStruct((2, _B, _QP), jnp.float32),
    compiler_params=pltpu.CompilerParams(
        use_tc_tiling_on_sc=False, needs_layout_passes=False
    ),
    scratch_types=[
        pltpu.VMEM((_C, _G * _L), jnp.float32),
        pltpu.VMEM((_G * _L,), jnp.float32),
        pltpu.VMEM((_G * _L,), jnp.float32),
    ],
)(_sc_body)


def _tc_body(bt_ref, sclb_ref, tsf_ref, conf_ref, *out_refs):
    bt = bt_ref[...]
    tsf = tsf_ref[...]
    cx = bt[:, 0, :]
    cy = bt[:, 1, :]
    w = bt[:, 2, :]
    h = bt[:, 3, :]
    img_h = tsf[:, 0:1]
    img_w = tsf[:, 1:2]
    y1 = (cy - 0.5 * h) * img_h
    x1 = (cx - 0.5 * w) * img_w
    y2 = (cy + 0.5 * h) * img_h
    x2 = (cx + 0.5 * w) * img_w
    sclb = sclb_ref[...]
    sc = sclb[0][:, :_Q]
    lb = sclb[1][:, :_Q]
    keep = sc > conf_ref[0, 0]
    for i in range(_B):
        leaf = jnp.stack([y1[i], x1[i], y2[i], x2[i], sc[i], lb[i]], axis=0)
        out_refs[i][...] = jnp.where(keep[i][None, :], leaf, 0.0)


def kernel(pred_logits, pred_boxes, target_sizes, confidence):
    lt = jnp.transpose(pred_logits, (2, 0, 1))
    lt = jnp.pad(lt, ((0, 0), (0, 0), (0, _QP - _Q)))
    sclb = _sc_softmax(lt)
    bt = jnp.transpose(pred_boxes, (0, 2, 1))
    tsf = target_sizes.astype(jnp.float32)
    conf = jnp.asarray(confidence, jnp.float32).reshape(1, 1)
    outs = pl.pallas_call(
        _tc_body,
        out_shape=tuple(
            jax.ShapeDtypeStruct((6, _Q), jnp.float32) for _ in range(_B)
        ),
    )(bt, sclb, tsf, conf)
    return tuple(jnp.transpose(o) for o in outs)


# speedup vs baseline: 2.5085x; 1.0013x over previous
import functools

import jax
import jax.numpy as jnp
from jax import lax
from jax.experimental import pallas as pl
from jax.experimental.pallas import tpu as pltpu
from jax.experimental.pallas import tpu_sc as plsc

_NC = 2
_NS = 16
_B = 16
_Q = 300
_QP = 304
_C = 92
_L = 16
_G = 10


def _combine(ma, la, mb, lb):
    m = jnp.maximum(ma, mb)
    l = jnp.where(mb > ma, lb, la)
    return m, jnp.where(mb == ma, jnp.minimum(la, lb), l)


def _sc_body(lt_hbm, sclb_hbm, lslab, sslab, llab):
    wid = lax.axis_index("s") * _NC + lax.axis_index("c")
    b = wid // 2
    half = wid % 2
    @pl.when(half == 0)
    def _():
        pltpu.sync_copy(lt_hbm.at[:, b, pl.ds(0, _G * _L)], lslab)

    @pl.when(half == 1)
    def _():
        pltpu.sync_copy(lt_hbm.at[:, b, pl.ds(144, _G * _L)], lslab)

    zeros = jnp.zeros((_L,), jnp.float32)
    neg = jnp.full((_L,), -jnp.inf, jnp.float32)

    def group(g, carry):
        qo = pl.multiple_of(g * _L, _L)

        def p1(i, acc):
            m0, m1, m2, m3, l0, l1, l2, l3, cf = acc
            c = i * 4
            v0 = lslab[c, pl.ds(qo, _L)]
            v1 = lslab[c + 1, pl.ds(qo, _L)]
            v2 = lslab[c + 2, pl.ds(qo, _L)]
            v3 = lslab[c + 3, pl.ds(qo, _L)]
            l0 = jnp.where(v0 > m0, cf, l0)
            l1 = jnp.where(v1 > m1, cf + 1.0, l1)
            l2 = jnp.where(v2 > m2, cf + 2.0, l2)
            l3 = jnp.where(v3 > m3, cf + 3.0, l3)
            return (
                jnp.maximum(m0, v0), jnp.maximum(m1, v1),
                jnp.maximum(m2, v2), jnp.maximum(m3, v3),
                l0, l1, l2, l3, cf + 4.0,
            )

        init = (neg, neg, neg, neg, zeros, zeros, zeros, zeros, zeros)
        m0, m1, m2, m3, l0, l1, l2, l3, cf = lax.fori_loop(0, 22, p1, init)
        v0 = lslab[88, pl.ds(qo, _L)]
        v1 = lslab[89, pl.ds(qo, _L)]
        v2 = lslab[90, pl.ds(qo, _L)]
        l0 = jnp.where(v0 > m0, cf, l0)
        l1 = jnp.where(v1 > m1, cf + 1.0, l1)
        l2 = jnp.where(v2 > m2, cf + 2.0, l2)
        m0 = jnp.maximum(m0, v0)
        m1 = jnp.maximum(m1, v1)
        m2 = jnp.maximum(m2, v2)
        ma, la = _combine(m0, l0, m1, l1)
        mb, lb = _combine(m2, l2, m3, l3)
        m91, lbl = _combine(ma, la, mb, lb)
        mall = jnp.maximum(m91, lslab[91, pl.ds(qo, _L)])

        def p2(i, acc):
            s0, s1, s2, s3 = acc
            c = i * 4
            return (
                s0 + jnp.exp(lslab[c, pl.ds(qo, _L)] - mall),
                s1 + jnp.exp(lslab[c + 1, pl.ds(qo, _L)] - mall),
                s2 + jnp.exp(lslab[c + 2, pl.ds(qo, _L)] - mall),
                s3 + jnp.exp(lslab[c + 3, pl.ds(qo, _L)] - mall),
            )

        s0, s1, s2, s3 = lax.fori_loop(0, 23, p2, (zeros, zeros, zeros, zeros))
        s = (s0 + s1) + (s2 + s3)
        sslab[pl.ds(qo, _L)] = jnp.exp(m91 - mall) / s
        llab[pl.ds(qo, _L)] = lbl
        return carry

    lax.fori_loop(0, _G, group, 0)

    @pl.when(half == 0)
    def _():
        pltpu.sync_copy(sslab, sclb_hbm.at[0, b, pl.ds(0, _G * _L)])
        pltpu.sync_copy(llab, sclb_hbm.at[1, b, pl.ds(0, _G * _L)])

    @pl.when(half == 1)
    def _():
        pltpu.sync_copy(sslab, sclb_hbm.at[0, b, pl.ds(144, _G * _L)])
        pltpu.sync_copy(llab, sclb_hbm.at[1, b, pl.ds(144, _G * _L)])


_sc_softmax = functools.partial(
    pl.kernel,
    mesh=plsc.VectorSubcoreMesh(core_axis_name="c", subcore_axis_name="s"),
    out_type=jax.ShapeDtypeStruct((2, _B, _QP), jnp.float32),
    compiler_params=pltpu.CompilerParams(
        use_tc_tiling_on_sc=False,
        needs_layout_passes=False,
        skip_device_barrier=True,
    ),
    scratch_types=[
        pltpu.VMEM((_C, _G * _L), jnp.float32),
        pltpu.VMEM((_G * _L,), jnp.float32),
        pltpu.VMEM((_G * _L,), jnp.float32),
    ],
)(_sc_body)


def _tc_body(bt_ref, sclb_ref, tsf_ref, conf_ref, *out_refs):
    bt = bt_ref[...]
    tsf = tsf_ref[...]
    cx = bt[:, 0, :]
    cy = bt[:, 1, :]
    w = bt[:, 2, :]
    h = bt[:, 3, :]
    img_h = tsf[:, 0:1]
    img_w = tsf[:, 1:2]
    y1 = (cy - 0.5 * h) * img_h
    x1 = (cx - 0.5 * w) * img_w
    y2 = (cy + 0.5 * h) * img_h
    x2 = (cx + 0.5 * w) * img_w
    sclb = sclb_ref[...]
    sc = sclb[0][:, :_Q]
    lb = sclb[1][:, :_Q]
    keep = sc > conf_ref[0, 0]
    for i in range(_B):
        leaf = jnp.stack([y1[i], x1[i], y2[i], x2[i], sc[i], lb[i]], axis=0)
        out_refs[i][...] = jnp.where(keep[i][None, :], leaf, 0.0)


def kernel(pred_logits, pred_boxes, target_sizes, confidence):
    lt = jnp.pad(pred_logits, ((0, 0), (0, _QP - _Q), (0, 0)))
    lt = jnp.transpose(lt, (2, 0, 1))
    sclb = _sc_softmax(lt)
    bt = jnp.transpose(pred_boxes, (0, 2, 1))
    tsf = target_sizes.astype(jnp.float32)
    conf = jnp.asarray(confidence, jnp.float32).reshape(1, 1)
    outs = pl.pallas_call(
        _tc_body,
        out_shape=tuple(
            jax.ShapeDtypeStruct((6, _Q), jnp.float32) for _ in range(_B)
        ),
    )(bt, sclb, tsf, conf)
    return tuple(jnp.transpose(o) for o in outs)


# speedup vs baseline: 2.7965x; 1.1148x over previous
import functools

import jax
import jax.numpy as jnp
from jax import lax
from jax.experimental import pallas as pl
from jax.experimental.pallas import tpu as pltpu
from jax.experimental.pallas import tpu_sc as plsc

_NC = 2
_NS = 16
_B = 16
_Q = 300
_QP = 384
_C = 92
_L = 16
_G = 10


def _combine(ma, la, mb, lb):
    m = jnp.maximum(ma, mb)
    l = jnp.where(mb > ma, lb, la)
    return m, jnp.where(mb == ma, jnp.minimum(la, lb), l)


def _sc_body(lt_hbm, sclb_hbm, lslab, sslab, llab):
    wid = lax.axis_index("s") * _NC + lax.axis_index("c")
    b = wid // 2
    half = wid % 2
    @pl.when(half == 0)
    def _():
        pltpu.sync_copy(lt_hbm.at[:, b, pl.ds(0, _G * _L)], lslab)

    @pl.when(half == 1)
    def _():
        pltpu.sync_copy(lt_hbm.at[:, b, pl.ds(144, _G * _L)], lslab)

    zeros = jnp.zeros((_L,), jnp.float32)
    neg = jnp.full((_L,), -jnp.inf, jnp.float32)

    def group(g, carry):
        qo = pl.multiple_of(g * _L, _L)

        def p1(i, acc):
            m0, m1, m2, m3, l0, l1, l2, l3, cf = acc
            c = i * 4
            v0 = lslab[c, pl.ds(qo, _L)]
            v1 = lslab[c + 1, pl.ds(qo, _L)]
            v2 = lslab[c + 2, pl.ds(qo, _L)]
            v3 = lslab[c + 3, pl.ds(qo, _L)]
            l0 = jnp.where(v0 > m0, cf, l0)
            l1 = jnp.where(v1 > m1, cf + 1.0, l1)
            l2 = jnp.where(v2 > m2, cf + 2.0, l2)
            l3 = jnp.where(v3 > m3, cf + 3.0, l3)
            return (
                jnp.maximum(m0, v0), jnp.maximum(m1, v1),
                jnp.maximum(m2, v2), jnp.maximum(m3, v3),
                l0, l1, l2, l3, cf + 4.0,
            )

        init = (neg, neg, neg, neg, zeros, zeros, zeros, zeros, zeros)
        m0, m1, m2, m3, l0, l1, l2, l3, cf = lax.fori_loop(0, 22, p1, init)
        v0 = lslab[88, pl.ds(qo, _L)]
        v1 = lslab[89, pl.ds(qo, _L)]
        v2 = lslab[90, pl.ds(qo, _L)]
        l0 = jnp.where(v0 > m0, cf, l0)
        l1 = jnp.where(v1 > m1, cf + 1.0, l1)
        l2 = jnp.where(v2 > m2, cf + 2.0, l2)
        m0 = jnp.maximum(m0, v0)
        m1 = jnp.maximum(m1, v1)
        m2 = jnp.maximum(m2, v2)
        ma, la = _combine(m0, l0, m1, l1)
        mb, lb = _combine(m2, l2, m3, l3)
        m91, lbl = _combine(ma, la, mb, lb)
        mall = jnp.maximum(m91, lslab[91, pl.ds(qo, _L)])

        def p2(i, acc):
            s0, s1, s2, s3 = acc
            c = i * 4
            return (
                s0 + jnp.exp(lslab[c, pl.ds(qo, _L)] - mall),
                s1 + jnp.exp(lslab[c + 1, pl.ds(qo, _L)] - mall),
                s2 + jnp.exp(lslab[c + 2, pl.ds(qo, _L)] - mall),
                s3 + jnp.exp(lslab[c + 3, pl.ds(qo, _L)] - mall),
            )

        s0, s1, s2, s3 = lax.fori_loop(0, 23, p2, (zeros, zeros, zeros, zeros))
        s = (s0 + s1) + (s2 + s3)
        sslab[pl.ds(qo, _L)] = jnp.exp(m91 - mall) / s
        llab[pl.ds(qo, _L)] = lbl
        return carry

    lax.fori_loop(0, _G, group, 0)

    @pl.when(half == 0)
    def _():
        pltpu.sync_copy(sslab, sclb_hbm.at[0, b, pl.ds(0, _G * _L)])
        pltpu.sync_copy(llab, sclb_hbm.at[1, b, pl.ds(0, _G * _L)])

    @pl.when(half == 1)
    def _():
        pltpu.sync_copy(sslab, sclb_hbm.at[0, b, pl.ds(144, _G * _L)])
        pltpu.sync_copy(llab, sclb_hbm.at[1, b, pl.ds(144, _G * _L)])


_sc_softmax = functools.partial(
    pl.kernel,
    mesh=plsc.VectorSubcoreMesh(core_axis_name="c", subcore_axis_name="s"),
    out_type=jax.ShapeDtypeStruct((2, _B, _QP), jnp.float32),
    compiler_params=pltpu.CompilerParams(
        use_tc_tiling_on_sc=False,
        needs_layout_passes=False,
        skip_device_barrier=True,
    ),
    scratch_types=[
        pltpu.VMEM((_C, _G * _L), jnp.float32),
        pltpu.VMEM((_G * _L,), jnp.float32),
        pltpu.VMEM((_G * _L,), jnp.float32),
    ],
)(_sc_body)


def _tc_body(bt_ref, sclb_ref, tsf_ref, conf_ref, *out_refs):
    bt = bt_ref[...]
    tsf = tsf_ref[...]
    cx = bt[:, 0, :]
    cy = bt[:, 1, :]
    w = bt[:, 2, :]
    h = bt[:, 3, :]
    img_h = tsf[:, 0:1]
    img_w = tsf[:, 1:2]
    y1 = (cy - 0.5 * h) * img_h
    x1 = (cx - 0.5 * w) * img_w
    y2 = (cy + 0.5 * h) * img_h
    x2 = (cx + 0.5 * w) * img_w
    sclb = sclb_ref[...]
    conf = conf_ref[0, 0]
    for i in range(_B):
        sc = jnp.reshape(sclb[3 * i:3 * i + 3, :], (_QP,))[:_Q]
        lb = jnp.reshape(sclb[48 + 3 * i:48 + 3 * i + 3, :], (_QP,))[:_Q]
        keep = sc > conf
        leaf = jnp.stack([y1[i], x1[i], y2[i], x2[i], sc, lb], axis=0)
        out_refs[i][...] = jnp.where(keep[None, :], leaf, 0.0)


def kernel(pred_logits, pred_boxes, target_sizes, confidence):
    lt = jnp.pad(pred_logits, ((0, 0), (0, _QP - _Q), (0, 0)))
    lt = jnp.transpose(lt, (2, 0, 1))
    sclb = _sc_softmax(lt).reshape(6 * _B, 128)
    bt = jnp.transpose(pred_boxes, (0, 2, 1))
    tsf = target_sizes.astype(jnp.float32)
    conf = jnp.asarray(confidence, jnp.float32).reshape(1, 1)
    outs = pl.pallas_call(
        _tc_body,
        out_shape=tuple(
            jax.ShapeDtypeStruct((6, _Q), jnp.float32) for _ in range(_B)
        ),
    )(bt, sclb, tsf, conf)
    return tuple(jnp.transpose(o) for o in outs)
